# baseline (speedup 1.0000x reference)
import jax
import jax.numpy as jnp
from jax import lax
from jax.experimental import pallas as pl
from jax.experimental.pallas import tpu as pltpu
from jax.experimental.pallas import tpu_sc as plsc

_D = 300
_B = 4096
_H = 50
_NC = 2
_NS = 16
_NW = _NC * _NS
_BPW = _B // _NW
_RC = 32
_QN = _BPW // _RC


def _embed_body(idx_hbm, table_hbm, out_hbm, idx_v, rows0, rows1, ob0, ob1,
                g0, g1, s0, s1):
    wid = lax.axis_index("s") * _NC + lax.axis_index("c")
    bb = wid * _BPW
    rows = (rows0, rows1)
    obs = (ob0, ob1)
    gs = (g0, g1)
    ss = (s0, s1)

    pltpu.sync_copy(
        idx_hbm.at[pl.ds(bb * _H, _BPW * _H)], idx_v.at[pl.ds(0, _BPW * _H)]
    )

    iota = lax.iota(jnp.int32, 16)
    zeros = iota - iota
    col_ids = [[iota + (q * _RC + ib * 16) for ib in range(2)]
               for q in range(_QN)]

    def fire(rslot, h, q):
        def fr(i, c):
            v = idx_v[pl.ds((q * _RC + i) * _H + h, 16)]
            pltpu.make_async_copy(
                table_hbm.at[v[0]], rows[rslot].at[i], gs[rslot]
            ).start()
            return c
        lax.fori_loop(0, _RC, fr, 0, unroll=8)

    def drain(rslot):
        def dr(i, c):
            pltpu.make_async_copy(
                table_hbm.at[0], rows[rslot].at[0], gs[rslot]
            ).wait()
            return c
        lax.fori_loop(0, _RC, dr, 0, unroll=16)

    def transpose(rslot, hslot, q):
        def td(d, c):
            dv = zeros + d
            for ib in range(2):
                g = plsc.load_gather(rows[rslot], [iota + ib * 16, dv])
                plsc.store_scatter(obs[hslot], [dv, col_ids[q][ib]], g)
            return c
        lax.fori_loop(0, _D, td, 0, unroll=4)

    def st_copy(hslot, h):
        return pltpu.make_async_copy(
            obs[hslot], out_hbm.at[h, :, pl.ds(bb, _BPW)], ss[hslot]
        )

    fire(0, 0, 0)

    def hpair(p, c):
        for hs in range(2):
            h = 2 * p + hs

            @pl.when(h >= 2)
            def _():
                st_copy(hs, h - 2).wait()

            for q in range(_QN):
                rs = q & 1
                if q < _QN - 1:
                    fire(1 - rs, h, q + 1)
                else:
                    @pl.when(h + 1 < _H)
                    def _():
                        fire(1 - rs, h + 1, 0)
                drain(rs)
                transpose(rs, hs, q)

            st_copy(hs, h).start()
        return c

    lax.fori_loop(0, _H // 2, hpair, 0)

    for hs in range(2):
        st_copy(hs, _H - 2 + hs).wait()


@jax.jit
def _embed_lookup(x_flat, table):
    mesh = plsc.VectorSubcoreMesh(core_axis_name="c", subcore_axis_name="s")
    run = pl.kernel(
        _embed_body,
        mesh=mesh,
        out_type=jax.ShapeDtypeStruct((_H, _D, _B), jnp.float32),
        scratch_types=[
            pltpu.VMEM((_BPW * _H + 16,), jnp.int32),
            pltpu.VMEM((_RC, _D), jnp.float32),
            pltpu.VMEM((_RC, _D), jnp.float32),
            pltpu.VMEM((_D, _BPW), jnp.float32),
            pltpu.VMEM((_D, _BPW), jnp.float32),
            pltpu.SemaphoreType.DMA,
            pltpu.SemaphoreType.DMA,
            pltpu.SemaphoreType.DMA,
            pltpu.SemaphoreType.DMA,
        ],
        compiler_params=pltpu.CompilerParams(needs_layout_passes=False),
    )
    return run(x_flat, table)


def kernel(x, table):
    x_flat = x.reshape(-1).astype(jnp.int32)
    out_p = _embed_lookup(x_flat, table)
    return jnp.transpose(out_p, (2, 0, 1))


# speedup vs baseline: 2.0893x; 2.0893x over previous
import jax
import jax.numpy as jnp
from jax import lax
from jax.experimental import pallas as pl
from jax.experimental.pallas import tpu as pltpu
from jax.experimental.pallas import tpu_sc as plsc

_D = 300
_B = 4096
_BH = _B // 2
_H = 50
_NC = 2
_NS = 16
_NW = _NC * _NS
_BPW = _BH // _NW
_CPB = 2
_CHUNK = _CPB * _H
_NCHUNKS = _BPW // _CPB


def _embed_body(idx_hbm, table_hbm, out_hbm, idx_v, rows_v, g0, g1, s0, s1):
    wid = lax.axis_index("s") * _NC + lax.axis_index("c")
    base = wid * _BPW * _H
    gsems = (g0, g1)
    ssems = (s0, s1)

    pltpu.sync_copy(
        idx_hbm.at[pl.ds(base, _BPW * _H)], idx_v.at[pl.ds(0, _BPW * _H)]
    )

    def out_copy(slot, c):
        b0 = wid * _BPW + c * _CPB
        return pltpu.make_async_copy(
            rows_v.at[slot], out_hbm.at[pl.ds(b0, _CPB)], ssems[slot]
        )

    def fire(slot, c):
        for jb in range(_CPB):
            def fire_row(jr, carry, _jb=jb):
                v = idx_v[pl.ds(c * _CHUNK + _jb * _H + jr, 16)]
                pltpu.make_async_copy(
                    table_hbm.at[v[0]], rows_v.at[slot, _jb, jr], gsems[slot]
                ).start()
                return carry
            lax.fori_loop(0, _H, fire_row, 0, unroll=10)

    def drain_gathers(slot):
        def drain(k, carry):
            pltpu.make_async_copy(
                table_hbm.at[0], rows_v.at[slot, 0, 0], gsems[slot]
            ).wait()
            return carry
        lax.fori_loop(0, _CHUNK, drain, 0, unroll=20)

    for slot in range(2):
        fire(slot, slot)

    def chunk_pair(p, carry):
        c0 = p * 2
        drain_gathers(0)
        out_copy(0, c0).start()
        drain_gathers(1)
        out_copy(1, c0 + 1).start()
        out_copy(0, c0).wait()
        fire(0, c0 + 2)
        out_copy(1, c0 + 1).wait()
        fire(1, c0 + 3)
        return carry

    lax.fori_loop(0, _NCHUNKS // 2 - 1, chunk_pair, 0)

    c0 = _NCHUNKS - 2
    drain_gathers(0)
    out_copy(0, c0).start()
    drain_gathers(1)
    out_copy(1, c0 + 1).start()
    out_copy(0, c0).wait()
    out_copy(1, c0 + 1).wait()


@jax.jit
def _embed_lookup(x_flat, table):
    mesh = plsc.VectorSubcoreMesh(core_axis_name="c", subcore_axis_name="s")
    run = pl.kernel(
        _embed_body,
        mesh=mesh,
        out_type=jax.ShapeDtypeStruct((_BH, _H, _D), jnp.float32),
        scratch_types=[
            pltpu.VMEM((_BPW * _H + 16,), jnp.int32),
            pltpu.VMEM((2, _CPB, _H, _D), jnp.float32),
            pltpu.SemaphoreType.DMA,
            pltpu.SemaphoreType.DMA,
            pltpu.SemaphoreType.DMA,
            pltpu.SemaphoreType.DMA,
        ],
    )
    o1 = run(x_flat[: _BH * _H], table)
    o2 = run(x_flat[_BH * _H:], table)
    return jnp.concatenate([o1, o2], axis=0)


def kernel(x, table):
    x_flat = x.reshape(-1).astype(jnp.int32)
    return _embed_lookup(x_flat, table)


# speedup vs baseline: 2.5550x; 1.2229x over previous
import jax
import jax.numpy as jnp
from jax import lax
from jax.experimental import pallas as pl
from jax.experimental.pallas import tpu as pltpu
from jax.experimental.pallas import tpu_sc as plsc

_D = 300
_B = 4096
_H = 50
_NC = 2
_NS = 16
_NW = _NC * _NS
_BPW = _B // _NW
_CPB = 2
_CHUNK = _CPB * _H
_NCHUNKS = _BPW // _CPB


def _embed_body(idx_hbm, table_hbm, out_hbm, idx_v, rows_v, g0, g1, s0, s1):
    wid = lax.axis_index("s") * _NC + lax.axis_index("c")
    base = wid * _BPW * _H
    gsems = (g0, g1)
    ssems = (s0, s1)

    pltpu.sync_copy(
        idx_hbm.at[pl.ds(base, _BPW * _H)], idx_v.at[pl.ds(0, _BPW * _H)]
    )

    def out_copy(slot, c):
        b0 = wid * _BPW + c * _CPB
        return pltpu.make_async_copy(
            rows_v.at[slot], out_hbm.at[pl.ds(b0, _CPB)], ssems[slot]
        )

    def fire(slot, c):
        for jb in range(_CPB):
            def fire_row(jr, carry, _jb=jb):
                v = idx_v[pl.ds(c * _CHUNK + _jb * _H + jr, 16)]
                pltpu.make_async_copy(
                    table_hbm.at[v[0]], rows_v.at[slot, _jb, jr], gsems[slot]
                ).start()
                return carry
            lax.fori_loop(0, _H, fire_row, 0, unroll=25)

    def drain_gathers(slot):
        def drain(k, carry):
            pltpu.make_async_copy(
                table_hbm.at[0], rows_v.at[slot, 0, 0], gsems[slot]
            ).wait()
            return carry
        lax.fori_loop(0, _CHUNK, drain, 0, unroll=25)

    for slot in range(2):
        fire(slot, slot)

    def chunk_pair(p, carry):
        c0 = p * 2
        drain_gathers(0)
        out_copy(0, c0).start()
        drain_gathers(1)
        out_copy(1, c0 + 1).start()
        out_copy(0, c0).wait()
        fire(0, c0 + 2)
        out_copy(1, c0 + 1).wait()
        fire(1, c0 + 3)
        return carry

    lax.fori_loop(0, _NCHUNKS // 2 - 1, chunk_pair, 0)

    c0 = _NCHUNKS - 2
    drain_gathers(0)
    out_copy(0, c0).start()
    drain_gathers(1)
    out_copy(1, c0 + 1).start()
    out_copy(0, c0).wait()
    out_copy(1, c0 + 1).wait()


@jax.jit
def _embed_lookup(x_flat, table):
    mesh = plsc.VectorSubcoreMesh(core_axis_name="c", subcore_axis_name="s")
    run = pl.kernel(
        _embed_body,
        mesh=mesh,
        out_type=jax.ShapeDtypeStruct((_B, _H, _D), jnp.float32),
        scratch_types=[
            pltpu.VMEM((_BPW * _H + 16,), jnp.int32),
            pltpu.VMEM((2, _CPB, _H, _D), jnp.float32),
            pltpu.SemaphoreType.DMA,
            pltpu.SemaphoreType.DMA,
            pltpu.SemaphoreType.DMA,
            pltpu.SemaphoreType.DMA,
        ],
    )
    return run(x_flat, table)


def kernel(x, table):
    x_flat = x.reshape(-1).astype(jnp.int32)
    return _embed_lookup(x_flat, table)
